# baseline (speedup 1.0000x reference)
import functools

import jax
import jax.numpy as jnp
from jax import lax
from jax.experimental import pallas as pl
from jax.experimental.pallas import tpu as pltpu
from jax.experimental.pallas import tpu_sc as plsc

VOCAB = 1000
D = 32
N = 4096 * 200

NC = 2
NS = 16
NW = NC * NS
PER_W = N // NW

C = 128
K = 10
CHUNK = C * K
N_STEP = PER_W // (2 * CHUNK)


@functools.partial(
    pl.kernel,
    out_type=jax.ShapeDtypeStruct((N, D), jnp.float32),
    mesh=plsc.VectorSubcoreMesh(
        core_axis_name="c", subcore_axis_name="s", num_cores=NC, num_subcores=NS
    ),
    scratch_types=[
        pltpu.VMEM((PER_W,), jnp.int32),
        pltpu.VMEM((CHUNK, D), jnp.float32),
        pltpu.VMEM((CHUNK, D), jnp.float32),
        pltpu.SemaphoreType.DMA,
        pltpu.SemaphoreType.DMA,
        pltpu.SemaphoreType.DMA,
    ],
    compiler_params=pltpu.CompilerParams(use_tc_tiling_on_sc=False),
)
def _gather_kernel(table_hbm, items_hbm, out_hbm, idx_v, rows0, rows1,
                   gsem0, gsem1, wsem):
    wid = lax.axis_index("s") * NC + lax.axis_index("c")
    base = wid * PER_W
    pltpu.sync_copy(items_hbm.at[pl.ds(base, PER_W)], idx_v)

    def body(i, _):
        j0 = 2 * i * CHUNK
        gd = []
        for rbuf, jb, gsem in ((rows0, j0, gsem0), (rows1, j0 + CHUNK, gsem1)):
            gd.append([
                pltpu.async_copy(
                    table_hbm.at[idx_v.at[pl.ds(jb + j * C, C)]],
                    rbuf.at[pl.ds(j * C, C)],
                    gsem,
                )
                for j in range(K)
            ])
        for d in gd[0]:
            d.wait()
        w0 = pltpu.async_copy(rows0, out_hbm.at[pl.ds(base + j0, CHUNK)], wsem)
        for d in gd[1]:
            d.wait()
        w1 = pltpu.async_copy(
            rows1, out_hbm.at[pl.ds(base + j0 + CHUNK, CHUNK)], wsem)
        w0.wait()
        w1.wait()
        return 0

    lax.fori_loop(0, N_STEP, body, 0)


def kernel(items, timesteps, item_embedding_table):
    del timesteps
    items_flat = items.reshape(-1).astype(jnp.int32)
    out = _gather_kernel(item_embedding_table, items_flat)
    return out.reshape(items.shape + (D,))
